# baseline (speedup 1.0000x reference)
import jax
import jax.numpy as jnp
from jax.experimental import pallas as pl
from jax.experimental.pallas import tpu as pltpu

_B = 64
_V = 100000
_ROW = _B - 1
_S = 8
_LC = 12544
_NC = 7
_W = _LC // _NC

_KS0 = 0
_KS1 = 42
_KS2 = _KS0 ^ _KS1 ^ 0x1BD11BDA
_ROTS = ((13, 15, 26, 6), (17, 29, 16, 24))


def _rotl(x, d):
    return (x << jnp.uint32(d)) | (x >> jnp.uint32(32 - d))


def _gumbel(flat):
    ks = (jnp.uint32(_KS0), jnp.uint32(_KS1), jnp.uint32(_KS2))
    x1_in = flat + jnp.uint32(_ROW * _V + _KS1)
    x0 = x1_in
    x1 = x1_in ^ _rotl(x1_in, _ROTS[0][0])
    for d in _ROTS[0][1:]:
        x0 = x0 + x1
        x1 = x0 ^ _rotl(x1, d)
    x0 = x0 + ks[1]
    x1 = x1 + ks[2] + jnp.uint32(1)
    for i in range(1, 5):
        for d in _ROTS[i % 2]:
            x0 = x0 + x1
            x1 = x0 ^ _rotl(x1, d)
        x0 = x0 + ks[(i + 1) % 3]
        x1 = x1 + ks[(i + 2) % 3] + jnp.uint32(i + 1)
    bits = x0 ^ x1
    fb = (bits >> jnp.uint32(9)) | jnp.uint32(0x3F800000)
    floats = jax.lax.bitcast_convert_type(fb, jnp.float32) - jnp.float32(1.0)
    tiny = jnp.float32(jnp.finfo(jnp.float32).tiny)
    u = jnp.maximum(tiny, floats * (jnp.float32(1.0) - tiny) + tiny)
    return -jnp.log(-jnp.log(u))


def _sample_kernel(temp_ref, logits_hbm, out_ref, blk_vmem, sem):
    cp = pltpu.make_async_copy(
        logits_hbm.at[pl.ds(_B - _S, _S), :], blk_vmem, sem)
    cp.start()
    inv_t = jnp.float32(1.0) / temp_ref[0]
    r = jax.lax.broadcasted_iota(jnp.uint32, (_S, _W), 0)
    c = jax.lax.broadcasted_iota(jnp.uint32, (_S, _W), 1)
    base = r * jnp.uint32(_LC) + c
    g0 = _gumbel(base)
    cp.wait()
    m_run = None
    for k in range(_NC):
        flat = base + jnp.uint32(k * _W)
        g = g0 if k == 0 else _gumbel(flat)
        parts = []
        for s in range(_S):
            lo = s * _LC + k * _W
            hi = lo + _W
            if hi <= _V:
                parts.append(blk_vmem[_S - 1 : _S, lo:hi])
            else:
                tail = blk_vmem[_S - 1 : _S, lo:_V]
                pad = jnp.zeros((1, hi - _V), jnp.float32)
                parts.append(jnp.concatenate([tail, pad], axis=1))
        row = jnp.concatenate(parts, axis=0)
        val = row * inv_t + g
        if k == _NC - 1:
            val = jnp.where(flat < jnp.uint32(_V), val,
                            jnp.float32(-jnp.inf))
        if m_run is None:
            m_run, best = val, flat
        else:
            take = val > m_run
            m_run = jnp.where(take, val, m_run)
            best = jnp.where(take, flat, best)
    m = jnp.max(m_run)
    idx = jnp.where(m_run == m, best.astype(jnp.int32),
                    jnp.int32(0x7FFFFFFF))
    out_ref[0, 0] = jnp.min(idx)


def kernel(logits, temperature):
    out = pl.pallas_call(
        _sample_kernel,
        out_shape=jax.ShapeDtypeStruct((1, 1), jnp.int32),
        in_specs=[
            pl.BlockSpec(memory_space=pltpu.SMEM),
            pl.BlockSpec(memory_space=pl.ANY),
        ],
        out_specs=pl.BlockSpec(memory_space=pltpu.SMEM),
        scratch_shapes=[
            pltpu.VMEM((_S, _V), jnp.float32),
            pltpu.SemaphoreType.DMA,
        ],
    )(temperature, logits)
    return out[0, 0]


# speedup vs baseline: 1.3397x; 1.3397x over previous
import jax
import jax.numpy as jnp
from jax.experimental import pallas as pl
from jax.experimental.pallas import tpu as pltpu

_B = 64
_V = 100000
_ROW = _B - 1
_S = 8
_LC = 12544
_NC = 7
_W = _LC // _NC

_KS0 = 0
_KS1 = 42
_KS2 = _KS0 ^ _KS1 ^ 0x1BD11BDA
_ROTS = ((13, 15, 26, 6), (17, 29, 16, 24))


def _rotl(x, d):
    return (x << jnp.uint32(d)) | (x >> jnp.uint32(32 - d))


def _gumbel(flat):
    ks = (jnp.uint32(_KS0), jnp.uint32(_KS1), jnp.uint32(_KS2))
    x1_in = flat + jnp.uint32(_ROW * _V + _KS1)
    x0 = x1_in
    x1 = x1_in ^ _rotl(x1_in, _ROTS[0][0])
    for d in _ROTS[0][1:]:
        x0 = x0 + x1
        x1 = x0 ^ _rotl(x1, d)
    x0 = x0 + ks[1]
    x1 = x1 + ks[2] + jnp.uint32(1)
    for i in range(1, 5):
        for d in _ROTS[i % 2]:
            x0 = x0 + x1
            x1 = x0 ^ _rotl(x1, d)
        x0 = x0 + ks[(i + 1) % 3]
        x1 = x1 + ks[(i + 2) % 3] + jnp.uint32(i + 1)
    bits = x0 ^ x1
    fb = (bits >> jnp.uint32(9)) | jnp.uint32(0x3F800000)
    floats = jax.lax.bitcast_convert_type(fb, jnp.float32) - jnp.float32(1.0)
    tiny = jnp.float32(jnp.finfo(jnp.float32).tiny)
    u = jnp.maximum(tiny, floats * (jnp.float32(1.0) - tiny) + tiny)
    return -jnp.log(-jnp.log(u))


def _sample_kernel(temp_ref, logits_hbm, out_ref, blk_vmem, sem):
    cp = pltpu.make_async_copy(
        logits_hbm.at[pl.ds(_B - _S, _S), :], blk_vmem, sem)
    cp.start()
    inv_t = jnp.float32(1.0) / temp_ref[0]
    r = jax.lax.broadcasted_iota(jnp.uint32, (_S, _W), 0)
    c = jax.lax.broadcasted_iota(jnp.uint32, (_S, _W), 1)
    base = r * jnp.uint32(_LC) + c
    gs = [_gumbel(base + jnp.uint32(k * _W)) for k in range(_NC)]
    cp.wait()
    m_run = None
    for k in range(_NC):
        flat = base + jnp.uint32(k * _W)
        g = gs[k]
        parts = []
        for s in range(_S):
            lo = s * _LC + k * _W
            hi = lo + _W
            if hi <= _V:
                parts.append(blk_vmem[_S - 1 : _S, lo:hi])
            else:
                tail = blk_vmem[_S - 1 : _S, lo:_V]
                pad = jnp.zeros((1, hi - _V), jnp.float32)
                parts.append(jnp.concatenate([tail, pad], axis=1))
        row = jnp.concatenate(parts, axis=0)
        val = row * inv_t + g
        if k == _NC - 1:
            val = jnp.where(flat < jnp.uint32(_V), val,
                            jnp.float32(-jnp.inf))
        if m_run is None:
            m_run, best = val, flat
        else:
            take = val > m_run
            m_run = jnp.where(take, val, m_run)
            best = jnp.where(take, flat, best)
    m = jnp.max(m_run)
    idx = jnp.where(m_run == m, best.astype(jnp.int32),
                    jnp.int32(0x7FFFFFFF))
    out_ref[0, 0] = jnp.min(idx)


def kernel(logits, temperature):
    out = pl.pallas_call(
        _sample_kernel,
        out_shape=jax.ShapeDtypeStruct((1, 1), jnp.int32),
        in_specs=[
            pl.BlockSpec(memory_space=pltpu.SMEM),
            pl.BlockSpec(memory_space=pl.ANY),
        ],
        out_specs=pl.BlockSpec(memory_space=pltpu.SMEM),
        scratch_shapes=[
            pltpu.VMEM((_S, _V), jnp.float32),
            pltpu.SemaphoreType.DMA,
        ],
    )(temperature, logits)
    return out[0, 0]


# speedup vs baseline: 1.3802x; 1.0302x over previous
import jax
import jax.numpy as jnp
from jax.experimental import pallas as pl
from jax.experimental.pallas import tpu as pltpu

_B = 64
_V = 100000
_ROW = _B - 1
_S = 8
_LC = 12544
_NC = 14
_W = _LC // _NC

_KS0 = 0
_KS1 = 42
_KS2 = _KS0 ^ _KS1 ^ 0x1BD11BDA
_ROTS = ((13, 15, 26, 6), (17, 29, 16, 24))


def _rotl(x, d):
    return (x << jnp.uint32(d)) | (x >> jnp.uint32(32 - d))


def _gumbel(flat):
    ks = (jnp.uint32(_KS0), jnp.uint32(_KS1), jnp.uint32(_KS2))
    x1_in = flat + jnp.uint32(_ROW * _V + _KS1)
    x0 = x1_in
    x1 = x1_in ^ _rotl(x1_in, _ROTS[0][0])
    for d in _ROTS[0][1:]:
        x0 = x0 + x1
        x1 = x0 ^ _rotl(x1, d)
    x0 = x0 + ks[1]
    x1 = x1 + ks[2] + jnp.uint32(1)
    for i in range(1, 5):
        for d in _ROTS[i % 2]:
            x0 = x0 + x1
            x1 = x0 ^ _rotl(x1, d)
        x0 = x0 + ks[(i + 1) % 3]
        x1 = x1 + ks[(i + 2) % 3] + jnp.uint32(i + 1)
    bits = x0 ^ x1
    fb = (bits >> jnp.uint32(9)) | jnp.uint32(0x3F800000)
    floats = jax.lax.bitcast_convert_type(fb, jnp.float32) - jnp.float32(1.0)
    u = jnp.maximum(floats, jnp.float32(jnp.finfo(jnp.float32).tiny))
    return -jnp.log(-jnp.log(u))


def _sample_kernel(temp_ref, logits_hbm, out_ref, blk_vmem, sem):
    cp = pltpu.make_async_copy(
        logits_hbm.at[pl.ds(_B - _S, _S), :], blk_vmem, sem)
    cp.start()
    inv_t = jnp.float32(1.0) / temp_ref[0]
    r = jax.lax.broadcasted_iota(jnp.uint32, (_S, _W), 0)
    c = jax.lax.broadcasted_iota(jnp.uint32, (_S, _W), 1)
    base = r * jnp.uint32(_LC) + c
    gs = [_gumbel(base + jnp.uint32(k * _W)) for k in range(_NC)]
    cp.wait()
    m_run = None
    for k in range(_NC):
        flat = base + jnp.uint32(k * _W)
        g = gs[k]
        parts = []
        for s in range(_S):
            lo = s * _LC + k * _W
            hi = lo + _W
            if hi <= _V:
                parts.append(blk_vmem[_S - 1 : _S, lo:hi])
            else:
                tail = blk_vmem[_S - 1 : _S, lo:_V]
                pad = jnp.zeros((1, hi - _V), jnp.float32)
                parts.append(jnp.concatenate([tail, pad], axis=1))
        row = jnp.concatenate(parts, axis=0)
        val = row * inv_t + g
        if k == _NC - 1:
            val = jnp.where(flat < jnp.uint32(_V), val,
                            jnp.float32(-jnp.inf))
        if m_run is None:
            m_run, best = val, flat
        else:
            take = val > m_run
            m_run = jnp.where(take, val, m_run)
            best = jnp.where(take, flat, best)
    m = jnp.max(m_run)
    idx = jnp.where(m_run == m, best.astype(jnp.int32),
                    jnp.int32(0x7FFFFFFF))
    out_ref[0, 0] = jnp.min(idx)


def kernel(logits, temperature):
    out = pl.pallas_call(
        _sample_kernel,
        out_shape=jax.ShapeDtypeStruct((1, 1), jnp.int32),
        in_specs=[
            pl.BlockSpec(memory_space=pltpu.SMEM),
            pl.BlockSpec(memory_space=pl.ANY),
        ],
        out_specs=pl.BlockSpec(memory_space=pltpu.SMEM),
        scratch_shapes=[
            pltpu.VMEM((_S, _V), jnp.float32),
            pltpu.SemaphoreType.DMA,
        ],
    )(temperature, logits)
    return out[0, 0]
